# baseline (speedup 1.0000x reference)
import functools

import jax
import jax.numpy as jnp
from jax import lax
from jax.experimental import pallas as pl
from jax.experimental.pallas import tpu as pltpu
from jax.experimental.pallas import tpu_sc as plsc

NC = 2
NS = 16
NW = NC * NS
LANES = 16

R = 1
NBUF = 8


@functools.lru_cache(maxsize=None)
def _build(rows, d):
    n_chunks = rows // (R * NW)
    assert rows % (R * NW) == 0 and n_chunks % NBUF == 0

    mesh = plsc.VectorSubcoreMesh(
        core_axis_name="c", subcore_axis_name="s", num_cores=NC, num_subcores=NS
    )

    @functools.partial(
        pl.kernel,
        out_type=jax.ShapeDtypeStruct((rows, d), jnp.float32),
        mesh=mesh,
        scratch_types=[
            pltpu.VMEM((LANES,), jnp.float32),
            [pltpu.VMEM((R, d), jnp.float32) for _ in range(4 * NBUF)],
            [pltpu.SemaphoreType.DMA for _ in range(2 * NBUF)],
        ],
    )
    def run(t_hbm, x_hbm, x1_hbm, m_hbm, out_hbm, tb, bufs, sems):
        xb = tuple(bufs[4 * b] for b in range(NBUF))
        x1b = tuple(bufs[4 * b + 1] for b in range(NBUF))
        mb = tuple(bufs[4 * b + 2] for b in range(NBUF))
        ob = tuple(bufs[4 * b + 3] for b in range(NBUF))
        in_sem = tuple(sems[:NBUF])
        out_sem = tuple(sems[NBUF:])

        wid = lax.axis_index("s") * NC + lax.axis_index("c")

        pltpu.sync_copy(t_hbm, tb)
        tv = tb[...]

        def chunk_row(c):
            return (c * NW + wid) * R

        def start_in(c, b):
            row = chunk_row(c)
            pltpu.async_copy(x_hbm.at[pl.ds(row, R), :], xb[b], in_sem[b])
            pltpu.async_copy(x1_hbm.at[pl.ds(row, R), :], x1b[b], in_sem[b])
            pltpu.async_copy(m_hbm.at[pl.ds(row, R), :], mb[b], in_sem[b])

        def wait_in(c, b):
            row = chunk_row(c)
            pltpu.make_async_copy(x_hbm.at[pl.ds(row, R), :], xb[b], in_sem[b]).wait()
            pltpu.make_async_copy(x_hbm.at[pl.ds(row, R), :], x1b[b], in_sem[b]).wait()
            pltpu.make_async_copy(x_hbm.at[pl.ds(row, R), :], mb[b], in_sem[b]).wait()

        for b in range(NBUF):
            start_in(b, b)

        @pl.loop(0, n_chunks, step=NBUF)
        def _blocks(i):
            for b in range(NBUF):
                c = i + b
                row = chunk_row(c)
                wait_in(c, b)

                @pl.when(i > 0)
                def _():
                    pltpu.make_async_copy(
                        ob[b], out_hbm.at[pl.ds(row, R), :], out_sem[b]
                    ).wait()

                xr, x1r, mr, orr = xb[b], x1b[b], mb[b], ob[b]

                for r in range(R):
                    @plsc.parallel_loop(0, d, step=LANES, unroll=8)
                    def _compute(j):
                        mv = mr[r, pl.ds(j, LANES)]
                        orr[r, pl.ds(j, LANES)] = jnp.where(
                            mv >= tv,
                            xr[r, pl.ds(j, LANES)],
                            x1r[r, pl.ds(j, LANES)],
                        )

                pltpu.async_copy(ob[b], out_hbm.at[pl.ds(row, R), :], out_sem[b])

                @pl.when(c + NBUF < n_chunks)
                def _():
                    start_in(c + NBUF, b)

        for b in range(NBUF):
            row = chunk_row(n_chunks - NBUF + b)
            pltpu.make_async_copy(
                ob[b], out_hbm.at[pl.ds(row, R), :], out_sem[b]
            ).wait()

    return run


def kernel(x, x1, mask, threshold):
    B, S, D = x.shape
    rows = B * S
    t16 = jnp.broadcast_to(threshold.astype(jnp.float32), (LANES,))
    run = _build(rows, D)
    out = run(t16, x.reshape(rows, D), x1.reshape(rows, D), mask.reshape(rows, D))
    return out.reshape(B, S, D)


# speedup vs baseline: 1.0018x; 1.0018x over previous
import functools

import jax
import jax.numpy as jnp
from jax import lax
from jax.experimental import pallas as pl
from jax.experimental.pallas import tpu as pltpu
from jax.experimental.pallas import tpu_sc as plsc

NC = 2
NS = 16
NW = NC * NS
LANES = 16

R = 2
NBUF = 4


@functools.lru_cache(maxsize=None)
def _build(rows, d):
    n_chunks = rows // (R * NW)
    assert rows % (R * NW) == 0 and n_chunks % NBUF == 0

    mesh = plsc.VectorSubcoreMesh(
        core_axis_name="c", subcore_axis_name="s", num_cores=NC, num_subcores=NS
    )

    @functools.partial(
        pl.kernel,
        out_type=jax.ShapeDtypeStruct((rows, d), jnp.float32),
        mesh=mesh,
        scratch_types=[
            pltpu.VMEM((LANES,), jnp.float32),
            [pltpu.VMEM((R, d), jnp.float32) for _ in range(4 * NBUF)],
            [pltpu.SemaphoreType.DMA for _ in range(2 * NBUF)],
        ],
    )
    def run(t_hbm, x_hbm, x1_hbm, m_hbm, out_hbm, tb, bufs, sems):
        xb = tuple(bufs[4 * b] for b in range(NBUF))
        x1b = tuple(bufs[4 * b + 1] for b in range(NBUF))
        mb = tuple(bufs[4 * b + 2] for b in range(NBUF))
        ob = tuple(bufs[4 * b + 3] for b in range(NBUF))
        in_sem = tuple(sems[:NBUF])
        out_sem = tuple(sems[NBUF:])

        wid = lax.axis_index("s") * NC + lax.axis_index("c")

        pltpu.sync_copy(t_hbm, tb)
        tv = tb[...]

        def chunk_row(c):
            return (c * NW + wid) * R

        def start_in(c, b):
            row = chunk_row(c)
            pltpu.async_copy(x_hbm.at[pl.ds(row, R), :], xb[b], in_sem[b])
            pltpu.async_copy(x1_hbm.at[pl.ds(row, R), :], x1b[b], in_sem[b])
            pltpu.async_copy(m_hbm.at[pl.ds(row, R), :], mb[b], in_sem[b])

        def wait_in(c, b):
            row = chunk_row(c)
            pltpu.make_async_copy(x_hbm.at[pl.ds(row, R), :], xb[b], in_sem[b]).wait()
            pltpu.make_async_copy(x_hbm.at[pl.ds(row, R), :], x1b[b], in_sem[b]).wait()
            pltpu.make_async_copy(x_hbm.at[pl.ds(row, R), :], mb[b], in_sem[b]).wait()

        for b in range(NBUF):
            start_in(b, b)

        @pl.loop(0, n_chunks, step=NBUF)
        def _blocks(i):
            for b in range(NBUF):
                c = i + b
                row = chunk_row(c)
                wait_in(c, b)

                @pl.when(i > 0)
                def _():
                    pltpu.make_async_copy(
                        ob[b], out_hbm.at[pl.ds(row, R), :], out_sem[b]
                    ).wait()

                xr, x1r, mr, orr = xb[b], x1b[b], mb[b], ob[b]

                for r in range(R):
                    @plsc.parallel_loop(0, d, step=LANES, unroll=8)
                    def _compute(j):
                        mv = mr[r, pl.ds(j, LANES)]
                        orr[r, pl.ds(j, LANES)] = jnp.where(
                            mv >= tv,
                            xr[r, pl.ds(j, LANES)],
                            x1r[r, pl.ds(j, LANES)],
                        )

                pltpu.async_copy(ob[b], out_hbm.at[pl.ds(row, R), :], out_sem[b])

                @pl.when(c + NBUF < n_chunks)
                def _():
                    start_in(c + NBUF, b)

        for b in range(NBUF):
            row = chunk_row(n_chunks - NBUF + b)
            pltpu.make_async_copy(
                ob[b], out_hbm.at[pl.ds(row, R), :], out_sem[b]
            ).wait()

    return run


def kernel(x, x1, mask, threshold):
    B, S, D = x.shape
    rows = B * S
    t16 = jnp.broadcast_to(threshold.astype(jnp.float32), (LANES,))
    run = _build(rows, D)
    out = run(t16, x.reshape(rows, D), x1.reshape(rows, D), mask.reshape(rows, D))
    return out.reshape(B, S, D)
